# baseline (speedup 1.0000x reference)
import jax
import jax.numpy as jnp
from jax import lax
from jax.experimental import pallas as pl
from jax.experimental.pallas import tpu as pltpu
from jax.experimental.pallas import tpu_sc as plsc

N = 10001
D = 128
E = 320000
NP = 10240
NC = 2
NS = 16
C = 125
SUP = 16
SUPE = SUP * C
EW1 = E // (NC * NS)
NSUP1 = EW1 // SUPE
EW2 = E // NS
NSUP2 = EW2 // SUPE
NPS = NP // NS
L = 16


def _sc_body(emb_p, row4a, col4a, row4b, col4b_lo, col4b_hi, vals, zeros,
             p_out, q_out, acc, rowb, colb, valb, g0, g1, sem0, sem1):
    c = lax.axis_index("c")
    s = lax.axis_index("s")
    w = c * NS + s
    rbase = s * NPS
    half = c * NP

    pltpu.sync_copy(zeros.at[pl.ds(rbase, NPS)], acc.at[pl.ds(rbase, NPS)])
    plsc.subcore_barrier()

    def spmm_phase(row4, col4_0, col4_1, widx, vbase, nsup, table):
        def sup(m, _):
            pltpu.sync_copy(row4.at[widx, m], rowb)
            if col4_1 is None:
                pltpu.sync_copy(col4_0.at[widx, m], colb)
            else:
                @pl.when(c == 0)
                def _():
                    pltpu.sync_copy(col4_0.at[widx, m], colb)

                @pl.when(c == 1)
                def _():
                    pltpu.sync_copy(col4_1.at[widx, m], colb)

            pltpu.sync_copy(vals.at[pl.ds(vbase + m * SUPE, SUPE)],
                            valb.at[pl.ds(0, SUPE)])

            def g_start(k, buf, sem):
                pltpu.async_copy(table.at[colb.at[k]], buf, sem)

            def g_wait(k, buf, sem):
                pltpu.make_async_copy(table.at[colb.at[k]], buf, sem).wait()

            def scale_scatter(k, buf):
                def edge(i, _):
                    vv = jnp.broadcast_to(valb[pl.ds(k * C + i, L)][0], (L,))
                    for j in range(D // L):
                        buf[i, pl.ds(j * L, L)] = (
                            buf[i, pl.ds(j * L, L)] * vv)
                    return 0

                lax.fori_loop(0, C, edge, 0, unroll=4)

                pltpu.sync_copy(buf, acc.at[rowb.at[k]], add=True)

            g_start(0, g0, sem0)

            def pair(t, _):
                k0 = 2 * t
                k1 = k0 + 1
                g_start(k1, g1, sem1)
                g_wait(k0, g0, sem0)
                scale_scatter(k0, g0)

                @pl.when(t < SUP // 2 - 1)
                def _():
                    g_start(k0 + 2, g0, sem0)

                g_wait(k1, g1, sem1)
                scale_scatter(k1, g1)
                return 0

            lax.fori_loop(0, SUP // 2, pair, 0)
            return 0

        lax.fori_loop(0, nsup, sup, 0)
        plsc.subcore_barrier()

    spmm_phase(row4a, col4a, None, w, w * EW1, NSUP1, emb_p)

    pltpu.sync_copy(acc.at[pl.ds(rbase, NPS)],
                    p_out.at[pl.ds(half + rbase, NPS)])
    pltpu.sync_copy(zeros.at[pl.ds(rbase, NPS)], acc.at[pl.ds(rbase, NPS)])
    plsc.subcore_barrier()

    spmm_phase(row4b, col4b_lo, col4b_hi, s, s * EW2, NSUP2, p_out)

    pltpu.sync_copy(acc.at[pl.ds(rbase, NPS)],
                    q_out.at[pl.ds(half + rbase, NPS)])


def _tc_body(e0_ref, p0_ref, p1_ref, q0_ref, q1_ref,
             e1_ref, e2_ref, sum_ref):
    e1 = p0_ref[...] + p1_ref[...]
    e2 = q0_ref[...] + q1_ref[...]
    e1_ref[...] = e1
    e2_ref[...] = e2
    sum_ref[...] = e0_ref[...] + e1 + e2


@jax.jit
def _run(emb_p, row4a, col4a, row4b, col4b_lo, col4b_hi, vals, zeros):
    mesh = plsc.VectorSubcoreMesh(core_axis_name="c", subcore_axis_name="s")
    sc = pl.kernel(
        _sc_body,
        out_type=(
            jax.ShapeDtypeStruct((NC * NP, D), jnp.float32),
            jax.ShapeDtypeStruct((NC * NP, D), jnp.float32),
        ),
        mesh=mesh,
        scratch_types=[
            pltpu.VMEM_SHARED((NP, D), jnp.float32),
            pltpu.VMEM((SUP, C), jnp.int32),
            pltpu.VMEM((SUP, C), jnp.int32),
            pltpu.VMEM((SUPE + L,), jnp.float32),
            pltpu.VMEM((C, D), jnp.float32),
            pltpu.VMEM((C, D), jnp.float32),
            pltpu.SemaphoreType.DMA,
            pltpu.SemaphoreType.DMA,
        ],
    )
    p_out, q_out = sc(emb_p, row4a, col4a, row4b, col4b_lo, col4b_hi,
                      vals, zeros)

    blk = 512
    grid = (NP // blk,)
    spec0 = pl.BlockSpec((blk, D), lambda i: (i, 0))
    spec1 = pl.BlockSpec((blk, D), lambda i: (i + NP // blk, 0))
    e1, e2, ssum = pl.pallas_call(
        _tc_body,
        grid=grid,
        in_specs=[spec0, spec0, spec1, spec0, spec1],
        out_specs=[spec0, spec0, spec0],
        out_shape=(
            jax.ShapeDtypeStruct((NP, D), jnp.float32),
            jax.ShapeDtypeStruct((NP, D), jnp.float32),
            jax.ShapeDtypeStruct((NP, D), jnp.float32),
        ),
    )(emb_p, p_out, p_out, q_out, q_out)
    return e1, e2, ssum


def kernel(edge_index, edge_values, item_emb):
    row = edge_index[0].astype(jnp.int32)
    col = edge_index[1].astype(jnp.int32)
    row4a = row.reshape(NC * NS, NSUP1, SUP, C)
    col4a = col.reshape(NC * NS, NSUP1, SUP, C)
    row4b = row.reshape(NS, NSUP2, SUP, C)
    col4b_lo = col.reshape(NS, NSUP2, SUP, C)
    col4b_hi = (col + NP).reshape(NS, NSUP2, SUP, C)

    emb_p = jnp.concatenate(
        [item_emb, jnp.zeros((NP - N, D), jnp.float32)], axis=0)
    zeros = jnp.zeros((NP, D), jnp.float32)

    e1, e2, ssum = _run(emb_p, row4a, col4a, row4b, col4b_lo, col4b_hi,
                        edge_values, zeros)
    return (ssum[:N], item_emb, e1[:N], e2[:N])


# speedup vs baseline: 1.1299x; 1.1299x over previous
import jax
import jax.numpy as jnp
from jax import lax
from jax.experimental import pallas as pl
from jax.experimental.pallas import tpu as pltpu
from jax.experimental.pallas import tpu_sc as plsc

N = 10001
D = 128
E = 320000
NP = 10240
NC = 2
NS = 16
C = 80
SUP = 25
SUPE = SUP * C
EW1 = E // (NC * NS)
NSUP1 = EW1 // SUPE
EW2 = E // NS
NSUP2 = EW2 // SUPE
NPS = NP // NS
L = 16


def _sc_body(emb_p, row4a, col4a, row4b, col4b_lo, col4b_hi, vals, zeros,
             p_out, q_out, acc, rowb, colb, valb, g0, g1, g2,
             gsem0, gsem1, gsem2, ssem0, ssem1, ssem2):
    c = lax.axis_index("c")
    s = lax.axis_index("s")
    w = c * NS + s
    rbase = s * NPS
    half = c * NP

    pltpu.sync_copy(zeros.at[pl.ds(rbase, NPS)], acc.at[pl.ds(rbase, NPS)])
    plsc.subcore_barrier()

    def spmm_phase(row4, col4_0, col4_1, widx, vbase, nsup, table):
        def sup(m, _):
            pltpu.sync_copy(row4.at[widx, m], rowb)
            if col4_1 is None:
                pltpu.sync_copy(col4_0.at[widx, m], colb)
            else:
                @pl.when(c == 0)
                def _():
                    pltpu.sync_copy(col4_0.at[widx, m], colb)

                @pl.when(c == 1)
                def _():
                    pltpu.sync_copy(col4_1.at[widx, m], colb)

            pltpu.sync_copy(vals.at[pl.ds(vbase + m * SUPE, SUPE)],
                            valb.at[pl.ds(0, SUPE)])

            ring = ((g0, gsem0, ssem0), (g1, gsem1, ssem1),
                    (g2, gsem2, ssem2))

            def g_start(k):
                buf, gsem, _ = ring[k % 3]
                pltpu.async_copy(table.at[colb.at[k]], buf, gsem)

            def g_wait(k):
                buf, gsem, _ = ring[k % 3]
                pltpu.make_async_copy(table.at[colb.at[k]], buf, gsem).wait()

            def s_start(k):
                buf, _, ssem = ring[k % 3]
                pltpu.async_copy(buf, acc.at[rowb.at[k]], ssem, add=True)

            def s_wait(k):
                buf, _, ssem = ring[k % 3]
                pltpu.make_async_copy(buf, acc.at[rowb.at[k]], ssem).wait()

            def scale(k):
                buf = ring[k % 3][0]

                def edge(i, _):
                    vv = jnp.broadcast_to(valb[pl.ds(k * C + i, L)][0], (L,))
                    for j in range(D // L):
                        buf[i, pl.ds(j * L, L)] = (
                            buf[i, pl.ds(j * L, L)] * vv)
                    return 0

                lax.fori_loop(0, C, edge, 0, unroll=4)

            g_start(0)
            g_start(1)
            for k in range(SUP):
                g_wait(k)
                scale(k)
                s_start(k)
                if k + 2 < SUP:
                    if k >= 1:
                        s_wait(k - 1)
                    g_start(k + 2)
            s_wait(SUP - 3)
            s_wait(SUP - 2)
            s_wait(SUP - 1)
            return 0

        lax.fori_loop(0, nsup, sup, 0)
        plsc.subcore_barrier()

    spmm_phase(row4a, col4a, None, w, w * EW1, NSUP1, emb_p)

    pltpu.sync_copy(acc.at[pl.ds(rbase, NPS)],
                    p_out.at[pl.ds(half + rbase, NPS)])
    pltpu.sync_copy(zeros.at[pl.ds(rbase, NPS)], acc.at[pl.ds(rbase, NPS)])
    plsc.subcore_barrier()

    spmm_phase(row4b, col4b_lo, col4b_hi, s, s * EW2, NSUP2, p_out)

    pltpu.sync_copy(acc.at[pl.ds(rbase, NPS)],
                    q_out.at[pl.ds(half + rbase, NPS)])


def _tc_body(e0_ref, p0_ref, p1_ref, q0_ref, q1_ref,
             e1_ref, e2_ref, sum_ref):
    e1 = p0_ref[...] + p1_ref[...]
    e2 = q0_ref[...] + q1_ref[...]
    e1_ref[...] = e1
    e2_ref[...] = e2
    sum_ref[...] = e0_ref[...] + e1 + e2


@jax.jit
def _run(emb_p, row4a, col4a, row4b, col4b_lo, col4b_hi, vals, zeros):
    mesh = plsc.VectorSubcoreMesh(core_axis_name="c", subcore_axis_name="s")
    sc = pl.kernel(
        _sc_body,
        out_type=(
            jax.ShapeDtypeStruct((NC * NP, D), jnp.float32),
            jax.ShapeDtypeStruct((NC * NP, D), jnp.float32),
        ),
        mesh=mesh,
        scratch_types=[
            pltpu.VMEM_SHARED((NP, D), jnp.float32),
            pltpu.VMEM((SUP, C), jnp.int32),
            pltpu.VMEM((SUP, C), jnp.int32),
            pltpu.VMEM((SUPE + L,), jnp.float32),
            pltpu.VMEM((C, D), jnp.float32),
            pltpu.VMEM((C, D), jnp.float32),
            pltpu.VMEM((C, D), jnp.float32),
            pltpu.SemaphoreType.DMA,
            pltpu.SemaphoreType.DMA,
            pltpu.SemaphoreType.DMA,
            pltpu.SemaphoreType.DMA,
            pltpu.SemaphoreType.DMA,
            pltpu.SemaphoreType.DMA,
        ],
    )
    p_out, q_out = sc(emb_p, row4a, col4a, row4b, col4b_lo, col4b_hi,
                      vals, zeros)

    blk = 512
    grid = (NP // blk,)
    spec0 = pl.BlockSpec((blk, D), lambda i: (i, 0))
    spec1 = pl.BlockSpec((blk, D), lambda i: (i + NP // blk, 0))
    e1, e2, ssum = pl.pallas_call(
        _tc_body,
        grid=grid,
        in_specs=[spec0, spec0, spec1, spec0, spec1],
        out_specs=[spec0, spec0, spec0],
        out_shape=(
            jax.ShapeDtypeStruct((NP, D), jnp.float32),
            jax.ShapeDtypeStruct((NP, D), jnp.float32),
            jax.ShapeDtypeStruct((NP, D), jnp.float32),
        ),
    )(emb_p, p_out, p_out, q_out, q_out)
    return e1, e2, ssum


def kernel(edge_index, edge_values, item_emb):
    row = edge_index[0].astype(jnp.int32)
    col = edge_index[1].astype(jnp.int32)
    row4a = row.reshape(NC * NS, NSUP1, SUP, C)
    col4a = col.reshape(NC * NS, NSUP1, SUP, C)
    row4b = row.reshape(NS, NSUP2, SUP, C)
    col4b_lo = col.reshape(NS, NSUP2, SUP, C)
    col4b_hi = (col + NP).reshape(NS, NSUP2, SUP, C)

    emb_p = jnp.concatenate(
        [item_emb, jnp.zeros((NP - N, D), jnp.float32)], axis=0)
    zeros = jnp.zeros((NP, D), jnp.float32)

    e1, e2, ssum = _run(emb_p, row4a, col4a, row4b, col4b_lo, col4b_hi,
                        edge_values, zeros)
    return (ssum[:N], item_emb, e1[:N], e2[:N])


# speedup vs baseline: 1.1732x; 1.0383x over previous
import jax
import jax.numpy as jnp
from jax import lax
from jax.experimental import pallas as pl
from jax.experimental.pallas import tpu as pltpu
from jax.experimental.pallas import tpu_sc as plsc

N = 10001
D = 128
E = 320000
NP = 10112
NC = 2
NS = 16
C = 80
SUP = 25
SUPE = SUP * C
EW1 = E // (NC * NS)
NSUP1 = EW1 // SUPE
EW2 = E // NS
NSUP2 = EW2 // SUPE
NPS = NP // NS
L = 16


def _sc_body(emb_p, row4a, col4a, row4b, col4b_lo, col4b_hi, vals, zeros,
             p_out, q_out, acc, rowb, colb, valb, g0, g1, g2,
             gsem0, gsem1, gsem2, ssem0, ssem1, ssem2, stsem):
    c = lax.axis_index("c")
    s = lax.axis_index("s")
    w = c * NS + s
    rbase = s * NPS
    half = c * NP

    pltpu.sync_copy(zeros.at[pl.ds(rbase, NPS)], acc.at[pl.ds(rbase, NPS)])
    plsc.subcore_barrier()

    def spmm_phase(row4, col4_0, col4_1, widx, vbase, nsup, table):
        def stage_ops(m, pty):
            yield row4.at[widx, m], rowb.at[pty]
            if col4_1 is None:
                yield col4_0.at[widx, m], colb.at[pty]
            else:
                yield col4_0.at[widx, m], colb.at[pty]


        def stage_start(m, pty):
            if col4_1 is None:
                for src, dst in stage_ops(m, pty):
                    pltpu.async_copy(src, dst, stsem)
            else:
                pltpu.async_copy(row4.at[widx, m], rowb.at[pty], stsem)

                @pl.when(c == 0)
                def _():
                    pltpu.async_copy(col4_0.at[widx, m], colb.at[pty],
                                     stsem)

                @pl.when(c == 1)
                def _():
                    pltpu.async_copy(col4_1.at[widx, m], colb.at[pty],
                                     stsem)


        def stage_wait(m, pty):
            for src, dst in stage_ops(m, pty):
                pltpu.make_async_copy(src, dst, stsem).wait()

        stage_start(0, 0)

        def sup(m, _):
            p = m % 2
            pltpu.sync_copy(vals.at[pl.ds(vbase + m * SUPE, SUPE)],
                            valb.at[pl.ds(0, SUPE)])
            stage_wait(m, p)

            @pl.when(m < nsup - 1)
            def _():
                stage_start(m + 1, 1 - p)

            ring = ((g0, gsem0, ssem0), (g1, gsem1, ssem1),
                    (g2, gsem2, ssem2))

            def g_start(k):
                buf, gsem, _ = ring[k % 3]
                pltpu.async_copy(table.at[colb.at[p, k]], buf, gsem)

            def g_wait(k):
                buf, gsem, _ = ring[k % 3]
                pltpu.make_async_copy(
                    table.at[colb.at[p, k]], buf, gsem).wait()

            def s_start(k):
                buf, _, ssem = ring[k % 3]
                pltpu.async_copy(buf, acc.at[rowb.at[p, k]], ssem, add=True)

            def s_wait(k):
                buf, _, ssem = ring[k % 3]
                pltpu.make_async_copy(
                    buf, acc.at[rowb.at[p, k]], ssem).wait()

            def scale(k):
                buf = ring[k % 3][0]

                def edge(i, _):
                    vv = jnp.broadcast_to(
                        valb[pl.ds(k * C + i, L)][0], (L,))
                    for j in range(D // L):
                        buf[i, pl.ds(j * L, L)] = (
                            buf[i, pl.ds(j * L, L)] * vv)
                    return 0

                lax.fori_loop(0, C, edge, 0, unroll=4)

            g_start(0)
            g_start(1)
            for k in range(SUP):
                g_wait(k)
                scale(k)
                s_start(k)
                if k + 2 < SUP:
                    if k >= 1:
                        s_wait(k - 1)
                    g_start(k + 2)
            s_wait(SUP - 3)
            s_wait(SUP - 2)
            s_wait(SUP - 1)
            return 0

        lax.fori_loop(0, nsup, sup, 0)
        plsc.subcore_barrier()

    spmm_phase(row4a, col4a, None, w, w * EW1, NSUP1, emb_p)

    pltpu.sync_copy(acc.at[pl.ds(rbase, NPS)],
                    p_out.at[pl.ds(half + rbase, NPS)])
    pltpu.sync_copy(zeros.at[pl.ds(rbase, NPS)], acc.at[pl.ds(rbase, NPS)])
    plsc.subcore_barrier()

    spmm_phase(row4b, col4b_lo, col4b_hi, s, s * EW2, NSUP2, p_out)

    pltpu.sync_copy(acc.at[pl.ds(rbase, NPS)],
                    q_out.at[pl.ds(half + rbase, NPS)])


def _tc_body(e0_ref, p0_ref, p1_ref, q0_ref, q1_ref,
             e1_ref, e2_ref, sum_ref):
    e1 = p0_ref[...] + p1_ref[...]
    e2 = q0_ref[...] + q1_ref[...]
    e1_ref[...] = e1
    e2_ref[...] = e2
    sum_ref[...] = e0_ref[...] + e1 + e2


@jax.jit
def _run(emb_p, row4a, col4a, row4b, col4b_lo, col4b_hi, vals, zeros):
    mesh = plsc.VectorSubcoreMesh(core_axis_name="c", subcore_axis_name="s")
    sc = pl.kernel(
        _sc_body,
        out_type=(
            jax.ShapeDtypeStruct((NC * NP, D), jnp.float32),
            jax.ShapeDtypeStruct((NC * NP, D), jnp.float32),
        ),
        mesh=mesh,
        scratch_types=[
            pltpu.VMEM_SHARED((NP, D), jnp.float32),
            pltpu.VMEM((2, SUP, C), jnp.int32),
            pltpu.VMEM((2, SUP, C), jnp.int32),
            pltpu.VMEM((SUPE + L,), jnp.float32),
            pltpu.VMEM((C, D), jnp.float32),
            pltpu.VMEM((C, D), jnp.float32),
            pltpu.VMEM((C, D), jnp.float32),
            pltpu.SemaphoreType.DMA,
            pltpu.SemaphoreType.DMA,
            pltpu.SemaphoreType.DMA,
            pltpu.SemaphoreType.DMA,
            pltpu.SemaphoreType.DMA,
            pltpu.SemaphoreType.DMA,
            pltpu.SemaphoreType.DMA,
        ],
    )
    p_out, q_out = sc(emb_p, row4a, col4a, row4b, col4b_lo, col4b_hi,
                      vals, zeros)

    blk = 632
    grid = (NP // blk,)
    spec0 = pl.BlockSpec((blk, D), lambda i: (i, 0))
    spec1 = pl.BlockSpec((blk, D), lambda i: (i + NP // blk, 0))
    e1, e2, ssum = pl.pallas_call(
        _tc_body,
        grid=grid,
        in_specs=[spec0, spec0, spec1, spec0, spec1],
        out_specs=[spec0, spec0, spec0],
        out_shape=(
            jax.ShapeDtypeStruct((NP, D), jnp.float32),
            jax.ShapeDtypeStruct((NP, D), jnp.float32),
            jax.ShapeDtypeStruct((NP, D), jnp.float32),
        ),
    )(emb_p, p_out, p_out, q_out, q_out)
    return e1, e2, ssum


def kernel(edge_index, edge_values, item_emb):
    row = edge_index[0].astype(jnp.int32)
    col = edge_index[1].astype(jnp.int32)
    row4a = row.reshape(NC * NS, NSUP1, SUP, C)
    col4a = col.reshape(NC * NS, NSUP1, SUP, C)
    row4b = row.reshape(NS, NSUP2, SUP, C)
    col4b_lo = col.reshape(NS, NSUP2, SUP, C)
    col4b_hi = (col + NP).reshape(NS, NSUP2, SUP, C)

    emb_p = jnp.concatenate(
        [item_emb, jnp.zeros((NP - N, D), jnp.float32)], axis=0)
    zeros = jnp.zeros((NP, D), jnp.float32)

    e1, e2, ssum = _run(emb_p, row4a, col4a, row4b, col4b_lo, col4b_hi,
                        edge_values, zeros)
    return (ssum[:N], item_emb, e1[:N], e2[:N])


# speedup vs baseline: 1.5410x; 1.3135x over previous
import jax
import jax.numpy as jnp
from jax import lax
from jax.experimental import pallas as pl
from jax.experimental.pallas import tpu as pltpu
from jax.experimental.pallas import tpu_sc as plsc

N = 10001
D = 128
E = 320000
NP = 10112
NC = 2
NS = 16
NW = NC * NS
C = 80
SUP = 25
SUPE = SUP * C
EW = E // NW
NSUP = EW // SUPE
NPS = NP // NS
L = 16


def _sc_body(table, row4, col4, vals, zeros, part_out,
             acc, rowb, colb, valb, g0, g1, g2,
             gsem0, gsem1, gsem2, ssem0, ssem1, ssem2, stsem):
    c = lax.axis_index("c")
    s = lax.axis_index("s")
    w = c * NS + s
    rbase = s * NPS
    half = c * NP

    pltpu.sync_copy(zeros.at[pl.ds(rbase, NPS)], acc.at[pl.ds(rbase, NPS)])
    plsc.subcore_barrier()

    def stage_ops(m, pty):
        yield row4.at[w, m], rowb.at[pty]
        yield col4.at[w, m], colb.at[pty]

    def stage_start(m, pty):
        for src, dst in stage_ops(m, pty):
            pltpu.async_copy(src, dst, stsem)

    def stage_wait(m, pty):
        for src, dst in stage_ops(m, pty):
            pltpu.make_async_copy(src, dst, stsem).wait()

    stage_start(0, 0)

    def sup(m, _):
        p = m % 2
        pltpu.sync_copy(vals.at[pl.ds(w * EW + m * SUPE, SUPE)],
                        valb.at[pl.ds(0, SUPE)])
        stage_wait(m, p)

        @pl.when(m < NSUP - 1)
        def _():
            stage_start(m + 1, 1 - p)

        ring = ((g0, gsem0, ssem0), (g1, gsem1, ssem1), (g2, gsem2, ssem2))

        def g_start(k):
            buf, gsem, _ = ring[k % 3]
            pltpu.async_copy(table.at[colb.at[p, k]], buf, gsem)

        def g_wait(k):
            buf, gsem, _ = ring[k % 3]
            pltpu.make_async_copy(table.at[colb.at[p, k]], buf, gsem).wait()

        def s_start(k):
            buf, _, ssem = ring[k % 3]
            pltpu.async_copy(buf, acc.at[rowb.at[p, k]], ssem, add=True)

        def s_wait(k):
            buf, _, ssem = ring[k % 3]
            pltpu.make_async_copy(buf, acc.at[rowb.at[p, k]], ssem).wait()

        def scale(k):
            buf = ring[k % 3][0]

            def edge(i, _):
                vv = jnp.broadcast_to(valb[pl.ds(k * C + i, L)][0], (L,))
                for j in range(D // L):
                    buf[i, pl.ds(j * L, L)] = buf[i, pl.ds(j * L, L)] * vv
                return 0

            lax.fori_loop(0, C, edge, 0, unroll=4)

        g_start(0)
        g_start(1)
        for k in range(SUP):
            g_wait(k)
            scale(k)
            s_start(k)
            if k + 2 < SUP:
                if k >= 1:
                    s_wait(k - 1)
                g_start(k + 2)
        s_wait(SUP - 3)
        s_wait(SUP - 2)
        s_wait(SUP - 1)
        return 0

    lax.fori_loop(0, NSUP, sup, 0)
    plsc.subcore_barrier()

    pltpu.sync_copy(acc.at[pl.ds(rbase, NPS)],
                    part_out.at[pl.ds(half + rbase, NPS)])


def _tc_merge_body(p0_ref, p1_ref, out_ref):
    out_ref[...] = p0_ref[...] + p1_ref[...]


def _tc_final_body(e0_ref, e1_ref, q0_ref, q1_ref, e2_ref, sum_ref):
    e2 = q0_ref[...] + q1_ref[...]
    e2_ref[...] = e2
    sum_ref[...] = e0_ref[...] + e1_ref[...] + e2


_BLK = 632
_SPEC0 = pl.BlockSpec((_BLK, D), lambda i: (i, 0))
_SPEC1 = pl.BlockSpec((_BLK, D), lambda i: (i + NP // _BLK, 0))


def _sc_layer(table, row4, col4, vals, zeros):
    mesh = plsc.VectorSubcoreMesh(core_axis_name="c", subcore_axis_name="s")
    sc = pl.kernel(
        _sc_body,
        out_type=jax.ShapeDtypeStruct((NC * NP, D), jnp.float32),
        mesh=mesh,
        scratch_types=[
            pltpu.VMEM_SHARED((NP, D), jnp.float32),
            pltpu.VMEM((2, SUP, C), jnp.int32),
            pltpu.VMEM((2, SUP, C), jnp.int32),
            pltpu.VMEM((SUPE + L,), jnp.float32),
            pltpu.VMEM((C, D), jnp.float32),
            pltpu.VMEM((C, D), jnp.float32),
            pltpu.VMEM((C, D), jnp.float32),
            pltpu.SemaphoreType.DMA,
            pltpu.SemaphoreType.DMA,
            pltpu.SemaphoreType.DMA,
            pltpu.SemaphoreType.DMA,
            pltpu.SemaphoreType.DMA,
            pltpu.SemaphoreType.DMA,
            pltpu.SemaphoreType.DMA,
        ],
    )
    return sc(table, row4, col4, vals, zeros)


@jax.jit
def _run(emb_p, row4, col4, vals, zeros):
    p_parts = _sc_layer(emb_p, row4, col4, vals, zeros)

    e1p = pl.pallas_call(
        _tc_merge_body,
        grid=(NP // _BLK,),
        in_specs=[_SPEC0, _SPEC1],
        out_specs=_SPEC0,
        out_shape=jax.ShapeDtypeStruct((NP, D), jnp.float32),
    )(p_parts, p_parts)

    q_parts = _sc_layer(e1p, row4, col4, vals, zeros)

    e2p, sump = pl.pallas_call(
        _tc_final_body,
        grid=(NP // _BLK,),
        in_specs=[_SPEC0, _SPEC0, _SPEC0, _SPEC1],
        out_specs=[_SPEC0, _SPEC0],
        out_shape=(
            jax.ShapeDtypeStruct((NP, D), jnp.float32),
            jax.ShapeDtypeStruct((NP, D), jnp.float32),
        ),
    )(emb_p, e1p, q_parts, q_parts)
    return e1p, e2p, sump


def kernel(edge_index, edge_values, item_emb):
    row = edge_index[0].astype(jnp.int32)
    col = edge_index[1].astype(jnp.int32)
    row4 = row.reshape(NW, NSUP, SUP, C)
    col4 = col.reshape(NW, NSUP, SUP, C)

    emb_p = jnp.concatenate(
        [item_emb, jnp.zeros((NP - N, D), jnp.float32)], axis=0)
    zeros = jnp.zeros((NP, D), jnp.float32)

    e1, e2, ssum = _run(emb_p, row4, col4, edge_values, zeros)
    return (ssum[:N], item_emb, e1[:N], e2[:N])


# speedup vs baseline: 1.5589x; 1.0116x over previous
import jax
import jax.numpy as jnp
from jax import lax
from jax.experimental import pallas as pl
from jax.experimental.pallas import tpu as pltpu
from jax.experimental.pallas import tpu_sc as plsc

N = 10001
D = 128
E = 320000
NP = 10112
NC = 2
NS = 16
NW = NC * NS
C = 80
SUP = 25
SUPE = SUP * C
EW = E // NW
NSUP = EW // SUPE
NPS = NP // NS
L = 16


def _sc_body(table, row4, col4, vals, zeros, part_out,
             acc, rowb, colb, valb, g0, g1, g2,
             gsem0, gsem1, gsem2, ssem0, ssem1, ssem2, stsem):
    c = lax.axis_index("c")
    s = lax.axis_index("s")
    w = c * NS + s
    rbase = s * NPS
    half = c * NP

    pltpu.sync_copy(zeros.at[pl.ds(rbase, NPS)], acc.at[pl.ds(rbase, NPS)])
    plsc.subcore_barrier()

    def stage_ops(m, pty):
        yield row4.at[w, m], rowb.at[pty]
        yield col4.at[w, m], colb.at[pty]

    def stage_start(m, pty):
        for src, dst in stage_ops(m, pty):
            pltpu.async_copy(src, dst, stsem)

    def stage_wait(m, pty):
        for src, dst in stage_ops(m, pty):
            pltpu.make_async_copy(src, dst, stsem).wait()

    stage_start(0, 0)

    def sup(m, _):
        p = m % 2
        pltpu.sync_copy(vals.at[pl.ds(w * EW + m * SUPE, SUPE)],
                        valb.at[pl.ds(0, SUPE)])
        stage_wait(m, p)

        @pl.when(m < NSUP - 1)
        def _():
            stage_start(m + 1, 1 - p)

        ring = ((g0, gsem0, ssem0), (g1, gsem1, ssem1), (g2, gsem2, ssem2))

        def g_start(k):
            buf, gsem, _ = ring[k % 3]
            pltpu.async_copy(table.at[colb.at[p, k]], buf, gsem)

        def g_wait(k):
            buf, gsem, _ = ring[k % 3]
            pltpu.make_async_copy(table.at[colb.at[p, k]], buf, gsem).wait()

        def s_start(k):
            buf, _, ssem = ring[k % 3]
            pltpu.async_copy(buf, acc.at[rowb.at[p, k]], ssem, add=True)

        def s_wait(k):
            buf, _, ssem = ring[k % 3]
            pltpu.make_async_copy(buf, acc.at[rowb.at[p, k]], ssem).wait()

        def scale(k):
            buf = ring[k % 3][0]

            def edge(i, _):
                vv = jnp.broadcast_to(valb[pl.ds(k * C + i, L)][0], (L,))
                for j in range(D // L):
                    buf[i, pl.ds(j * L, L)] = buf[i, pl.ds(j * L, L)] * vv
                return 0

            lax.fori_loop(0, C, edge, 0, unroll=4)

        g_start(0)
        g_start(1)
        for k in range(SUP):
            g_wait(k)
            scale(k)
            s_start(k)
            if k + 2 < SUP:
                if k >= 1:
                    s_wait(k - 1)
                g_start(k + 2)
        s_wait(SUP - 3)
        s_wait(SUP - 2)
        s_wait(SUP - 1)
        return 0

    lax.fori_loop(0, NSUP, sup, 0)
    plsc.subcore_barrier()

    pltpu.sync_copy(acc.at[pl.ds(rbase, NPS)],
                    part_out.at[pl.ds(half + rbase, NPS)])


def _tc_merge_body(p0_ref, p1_ref, out_ref):
    out_ref[...] = p0_ref[...] + p1_ref[...]


def _tc_final_body(e0_ref, e1_ref, q0_ref, q1_ref, e2_ref, sum_ref):
    e2 = q0_ref[...] + q1_ref[...]
    e2_ref[...] = e2
    sum_ref[...] = e0_ref[...] + e1_ref[...] + e2


_BLK = 632
_SPEC0 = pl.BlockSpec((_BLK, D), lambda i: (i, 0))
_SPEC1 = pl.BlockSpec((_BLK, D), lambda i: (i + NP // _BLK, 0))


def _sc_layer(table, row4, col4, vals, zeros):
    mesh = plsc.VectorSubcoreMesh(core_axis_name="c", subcore_axis_name="s")
    sc = pl.kernel(
        _sc_body,
        out_type=jax.ShapeDtypeStruct((NC * NP, D), jnp.float32),
        mesh=mesh,
        scratch_types=[
            pltpu.VMEM_SHARED((NP, D), jnp.float32),
            pltpu.VMEM((2, SUP, C), jnp.int32),
            pltpu.VMEM((2, SUP, C), jnp.int32),
            pltpu.VMEM((SUPE + L,), jnp.float32),
            pltpu.VMEM((C, D), jnp.float32),
            pltpu.VMEM((C, D), jnp.float32),
            pltpu.VMEM((C, D), jnp.float32),
            pltpu.SemaphoreType.DMA,
            pltpu.SemaphoreType.DMA,
            pltpu.SemaphoreType.DMA,
            pltpu.SemaphoreType.DMA,
            pltpu.SemaphoreType.DMA,
            pltpu.SemaphoreType.DMA,
            pltpu.SemaphoreType.DMA,
        ],
    )
    return sc(table, row4, col4, vals, zeros)


@jax.jit
def _run(emb_p, row4, col4, vals, zeros):
    p_parts = _sc_layer(emb_p, row4, col4, vals, zeros)

    e1p = p_parts[:NP] + p_parts[NP:]

    q_parts = _sc_layer(e1p, row4, col4, vals, zeros)

    e2p, sump = pl.pallas_call(
        _tc_final_body,
        grid=(NP // _BLK,),
        in_specs=[_SPEC0, _SPEC0, _SPEC0, _SPEC1],
        out_specs=[_SPEC0, _SPEC0],
        out_shape=(
            jax.ShapeDtypeStruct((NP, D), jnp.float32),
            jax.ShapeDtypeStruct((NP, D), jnp.float32),
        ),
    )(emb_p, e1p, q_parts, q_parts)
    return e1p, e2p, sump


def kernel(edge_index, edge_values, item_emb):
    row = edge_index[0].astype(jnp.int32)
    col = edge_index[1].astype(jnp.int32)
    row4 = row.reshape(NW, NSUP, SUP, C)
    col4 = col.reshape(NW, NSUP, SUP, C)

    emb_p = jnp.concatenate(
        [item_emb, jnp.zeros((NP - N, D), jnp.float32)], axis=0)
    zeros = jnp.zeros((NP, D), jnp.float32)

    e1, e2, ssum = _run(emb_p, row4, col4, edge_values, zeros)
    return (ssum[:N], item_emb, e1[:N], e2[:N])


# speedup vs baseline: 1.6214x; 1.0401x over previous
import jax
import jax.numpy as jnp
from jax import lax
from jax.experimental import pallas as pl
from jax.experimental.pallas import tpu as pltpu
from jax.experimental.pallas import tpu_sc as plsc

N = 10001
D = 128
E = 320000
NP = 10112
NC = 2
NS = 16
NW = NC * NS
C = 80
SUP = 25
SUPE = SUP * C
EW = E // NW
NSUP = EW // SUPE
NPS = NP // NS
L = 16


def _sc_body(table, row4, col4, vals, zeros, part_out,
             acc, rowb, colb, valb, g0, g1, g2,
             gsem0, gsem1, gsem2, ssem0, ssem1, ssem2, stsem):
    c = lax.axis_index("c")
    s = lax.axis_index("s")
    w = c * NS + s
    rbase = s * NPS
    half = c * NP

    pltpu.sync_copy(zeros.at[pl.ds(rbase, NPS)], acc.at[pl.ds(rbase, NPS)])
    plsc.subcore_barrier()

    def stage_ops(m, pty):
        yield row4.at[w, m], rowb.at[pty]
        yield col4.at[w, m], colb.at[pty]

    def stage_start(m, pty):
        for src, dst in stage_ops(m, pty):
            pltpu.async_copy(src, dst, stsem)

    def stage_wait(m, pty):
        for src, dst in stage_ops(m, pty):
            pltpu.make_async_copy(src, dst, stsem).wait()

    stage_start(0, 0)

    def sup(m, _):
        p = m % 2
        pltpu.sync_copy(vals.at[pl.ds(w * EW + m * SUPE, SUPE)],
                        valb.at[pl.ds(0, SUPE)])
        stage_wait(m, p)

        @pl.when(m < NSUP - 1)
        def _():
            stage_start(m + 1, 1 - p)

        ring = ((g0, gsem0, ssem0), (g1, gsem1, ssem1), (g2, gsem2, ssem2))

        def g_start(k):
            buf, gsem, _ = ring[k % 3]
            pltpu.async_copy(table.at[colb.at[p, k]], buf, gsem)

        def g_wait(k):
            buf, gsem, _ = ring[k % 3]
            pltpu.make_async_copy(table.at[colb.at[p, k]], buf, gsem).wait()

        def s_start(k):
            buf, _, ssem = ring[k % 3]
            pltpu.async_copy(buf, acc.at[rowb.at[p, k]], ssem, add=True)

        def s_wait(k):
            buf, _, ssem = ring[k % 3]
            pltpu.make_async_copy(buf, acc.at[rowb.at[p, k]], ssem).wait()

        def scale(k):
            buf = ring[k % 3][0]

            def edge(i, _):
                vv = jnp.broadcast_to(valb[pl.ds(k * C + i, L)][0], (L,))
                for j in range(D // L):
                    buf[i, pl.ds(j * L, L)] = buf[i, pl.ds(j * L, L)] * vv
                return 0

            lax.fori_loop(0, C, edge, 0, unroll=4)

        g_start(0)
        g_start(1)
        for k in range(SUP):
            g_wait(k)
            scale(k)
            s_start(k)
            if k + 2 < SUP:
                if k >= 1:
                    s_wait(k - 1)
                g_start(k + 2)
        s_wait(SUP - 3)
        s_wait(SUP - 2)
        s_wait(SUP - 1)
        return 0

    lax.fori_loop(0, NSUP, sup, 0)
    plsc.subcore_barrier()

    pltpu.sync_copy(acc.at[pl.ds(rbase, NPS)],
                    part_out.at[pl.ds(half + rbase, NPS)])


def _tc_merge_body(p0_ref, p1_ref, out_ref):
    out_ref[...] = p0_ref[...] + p1_ref[...]


def _tc_final_body(e0_ref, e1_ref, q0_ref, q1_ref, e1o_ref, e2_ref,
                   sum_ref):
    e1 = e1_ref[...]
    e2 = q0_ref[...] + q1_ref[...]
    e1o_ref[...] = e1
    e2_ref[...] = e2
    sum_ref[...] = e0_ref[...] + e1 + e2


_BLK = 632
_SPEC0 = pl.BlockSpec((_BLK, D), lambda i: (i, 0))
_SPEC1 = pl.BlockSpec((_BLK, D), lambda i: (i + NP // _BLK, 0))


def _sc_layer(table, row4, col4, vals, zeros):
    mesh = plsc.VectorSubcoreMesh(core_axis_name="c", subcore_axis_name="s")
    sc = pl.kernel(
        _sc_body,
        out_type=jax.ShapeDtypeStruct((NC * NP, D), jnp.float32),
        mesh=mesh,
        scratch_types=[
            pltpu.VMEM_SHARED((NP, D), jnp.float32),
            pltpu.VMEM((2, SUP, C), jnp.int32),
            pltpu.VMEM((2, SUP, C), jnp.int32),
            pltpu.VMEM((SUPE + L,), jnp.float32),
            pltpu.VMEM((C, D), jnp.float32),
            pltpu.VMEM((C, D), jnp.float32),
            pltpu.VMEM((C, D), jnp.float32),
            pltpu.SemaphoreType.DMA,
            pltpu.SemaphoreType.DMA,
            pltpu.SemaphoreType.DMA,
            pltpu.SemaphoreType.DMA,
            pltpu.SemaphoreType.DMA,
            pltpu.SemaphoreType.DMA,
            pltpu.SemaphoreType.DMA,
        ],
    )
    return sc(table, row4, col4, vals, zeros)


@jax.jit
def _run(emb, row4, col4, vals, zeros):
    p_parts = _sc_layer(emb, row4, col4, vals, zeros)

    e1 = p_parts[:N] + p_parts[NP:NP + N]

    q_parts = _sc_layer(e1, row4, col4, vals, zeros)

    e1o, e2, ssum = pl.pallas_call(
        _tc_final_body,
        grid=(pl.cdiv(N, _BLK),),
        in_specs=[_SPEC0, _SPEC0, _SPEC0, _SPEC1],
        out_specs=[_SPEC0, _SPEC0, _SPEC0],
        out_shape=(
            jax.ShapeDtypeStruct((N, D), jnp.float32),
            jax.ShapeDtypeStruct((N, D), jnp.float32),
            jax.ShapeDtypeStruct((N, D), jnp.float32),
        ),
    )(emb, e1, q_parts, q_parts)
    return e1o, e2, ssum


def kernel(edge_index, edge_values, item_emb):
    row = edge_index[0].astype(jnp.int32)
    col = edge_index[1].astype(jnp.int32)
    row4 = row.reshape(NW, NSUP, SUP, C)
    col4 = col.reshape(NW, NSUP, SUP, C)
    zeros = jnp.zeros((NP, D), jnp.float32)

    e1, e2, ssum = _run(item_emb, row4, col4, edge_values, zeros)
    return (ssum, item_emb, e1, e2)
